# Initial kernel scaffold; baseline (speedup 1.0000x reference)
import jax
import jax.numpy as jnp
from jax.experimental import pallas as pl


def kernel(pred_logits, pred_boxes, tgt_boxes, tgt_labels):
    raise NotImplementedError("write your pallas kernel here")


# speedup vs baseline: 4.0704x; 4.0704x over previous
import jax
import jax.numpy as jnp
from jax.experimental import pallas as pl
from jax.experimental.pallas import tpu as pltpu

BS, Q, C1, T = 64, 1000, 1001, 100
NUM_CLASSES = C1 - 1


def _loss_kernel(tl_ref, logits_ref, pb_ref, tb_ref, out_ref):
    b = pl.program_id(0)

    x = logits_ref[0]
    m = jnp.max(x, axis=-1)
    lse = m + jnp.log(jnp.sum(jnp.exp(x - m[:, None]), axis=-1))
    s_lse = jnp.sum(lse)

    cn = x[:, NUM_CLASSES:NUM_CLASSES + 1]
    unmatched = jax.lax.broadcasted_iota(jnp.int32, (Q, 1), 0) >= T
    s_no = jnp.sum(jnp.where(unmatched, cn, 0.0))

    labels = tl_ref[0, 0]
    oh = jax.lax.broadcasted_iota(jnp.int32, (T, C1), 1) == labels[:, None]
    s_m = jnp.sum(jnp.where(oh, x[:T, :], 0.0))

    l1 = jnp.sum(jnp.abs(pb_ref[0, :T, :] - tb_ref[0]))

    @pl.when(b == 0)
    def _():
        out_ref[0] = 0.0
        out_ref[1] = 0.0

    out_ref[0] += s_lse - s_no - s_m
    out_ref[1] += l1

    @pl.when(b == BS - 1)
    def _():
        out_ref[0] = out_ref[0] / (BS * Q)
        out_ref[1] = out_ref[1] / (BS * T * 4)


def kernel(pred_logits, pred_boxes, tgt_boxes, tgt_labels):
    tl3 = tgt_labels.astype(jnp.int32).reshape(BS, 1, T)
    out = pl.pallas_call(
        _loss_kernel,
        grid=(BS,),
        in_specs=[
            pl.BlockSpec((1, 1, T), lambda b: (b, 0, 0)),
            pl.BlockSpec((1, Q, C1), lambda b: (b, 0, 0)),
            pl.BlockSpec((1, Q, 4), lambda b: (b, 0, 0)),
            pl.BlockSpec((1, T, 4), lambda b: (b, 0, 0)),
        ],
        out_specs=pl.BlockSpec(memory_space=pltpu.SMEM),
        out_shape=jax.ShapeDtypeStruct((2,), jnp.float32),
    )(tl3, pred_logits, pred_boxes, tgt_boxes)
    return out


# speedup vs baseline: 4.8756x; 1.1978x over previous
import jax
import jax.numpy as jnp
from jax.experimental import pallas as pl
from jax.experimental.pallas import tpu as pltpu

BS, Q, C1, T = 64, 1000, 1001, 100
NUM_CLASSES = C1 - 1


BB = 4


def _loss_kernel(tl_ref, logits_ref, pb_ref, tb_ref, out_ref):
    b = pl.program_id(0)

    s_lse = 0.0
    s_g = 0.0
    l1 = 0.0
    for i in range(BB):
        x = logits_ref[i]
        m = jnp.max(x, axis=-1)
        lse = m + jnp.log(jnp.sum(jnp.exp(x - m[:, None]), axis=-1))
        s_lse += jnp.sum(lse)

        cn = x[:, NUM_CLASSES:NUM_CLASSES + 1]
        unmatched = jax.lax.broadcasted_iota(jnp.int32, (Q, 1), 0) >= T
        s_g += jnp.sum(jnp.where(unmatched, cn, 0.0))

        labels = tl_ref[i, 0]
        oh = jax.lax.broadcasted_iota(jnp.int32, (T, C1), 1) == labels[:, None]
        s_g += jnp.sum(jnp.where(oh, x[:T, :], 0.0))

        l1 += jnp.sum(jnp.abs(pb_ref[i, :T, :] - tb_ref[i]))

    @pl.when(b == 0)
    def _():
        out_ref[0] = 0.0
        out_ref[1] = 0.0

    out_ref[0] += s_lse - s_g
    out_ref[1] += l1

    @pl.when(b == BS // BB - 1)
    def _():
        out_ref[0] = out_ref[0] / (BS * Q)
        out_ref[1] = out_ref[1] / (BS * T * 4)


def kernel(pred_logits, pred_boxes, tgt_boxes, tgt_labels):
    tl3 = tgt_labels.astype(jnp.int32).reshape(BS, 1, T)
    out = pl.pallas_call(
        _loss_kernel,
        grid=(BS // BB,),
        in_specs=[
            pl.BlockSpec((BB, 1, T), lambda b: (b, 0, 0)),
            pl.BlockSpec((BB, Q, C1), lambda b: (b, 0, 0)),
            pl.BlockSpec((BB, Q, 4), lambda b: (b, 0, 0)),
            pl.BlockSpec((BB, T, 4), lambda b: (b, 0, 0)),
        ],
        out_specs=pl.BlockSpec(memory_space=pltpu.SMEM),
        out_shape=jax.ShapeDtypeStruct((2,), jnp.float32),
    )(tl3, pred_logits, pred_boxes, tgt_boxes)
    return out
